# baseline (speedup 1.0000x reference)
import functools

import jax
import jax.numpy as jnp
from jax import lax
from jax.experimental import pallas as pl
from jax.experimental.pallas import tpu as pltpu
from jax.experimental.pallas import tpu_sc as plsc

_B, _N, _F = 16, 10000, 128
_E = 256
_NO = _N + _E
_HB = _NO // 2
_W = 32
_PW = (_B * _E) // _W


def _pool_body(x_hbm, idx_hbm, out_hbm, bufa, bufb, bufc, idx_v, psem, ssem):
    w = lax.axis_index("s") * 2 + lax.axis_index("c")
    b = w // 2
    h = w % 2
    src = b * _N + 2 * _PW * h
    pltpu.async_copy(x_hbm.at[pl.ds(src, _PW), :], bufa, psem)
    pltpu.async_copy(x_hbm.at[pl.ds(src + _PW, _PW), :], bufb, psem)
    for _ in range(2):
        pltpu.make_async_copy(x_hbm.at[pl.ds(0, _PW), :], bufa, psem).wait()

    half = _PW // 2

    def _avg_span(lo, buf):
        def _pair_avg(e, carry):
            for c_ in range(8):
                sl = pl.ds(c_ * 16, 16)
                bufc[lo + e, sl] = 0.5 * (buf[2 * e, sl] + buf[2 * e + 1, sl])
            return carry

        lax.fori_loop(0, half, _pair_avg, 0)

    _avg_span(0, bufa)
    s0 = pltpu.make_async_copy(
        bufc.at[pl.ds(0, half), :], out_hbm.at[pl.ds(_PW * w, half), :], ssem
    )
    s0.start()
    _avg_span(half, bufb)
    s1 = pltpu.make_async_copy(
        bufc.at[pl.ds(half, half), :],
        out_hbm.at[pl.ds(_PW * w + half, half), :],
        ssem,
    )
    s1.start()
    s0.wait()
    s1.wait()


_pool_call = functools.partial(
    pl.kernel,
    out_type=jax.ShapeDtypeStruct((_B * _E, _F), jnp.float32),
    mesh=plsc.VectorSubcoreMesh(core_axis_name="c", subcore_axis_name="s"),
    scratch_types=[
        pltpu.VMEM((_PW, _F), jnp.float32),
        pltpu.VMEM((_PW, _F), jnp.float32),
        pltpu.VMEM((_PW, _F), jnp.float32),
        pltpu.VMEM((2, 128), jnp.int32),
        pltpu.SemaphoreType.DMA,
        pltpu.SemaphoreType.DMA,
    ],
)(_pool_body)


def _copy_body(in_ref, out_ref):
    out_ref[...] = in_ref[...]


def kernel(inputs, pool_idx):
    x_flat = inputs.reshape(_B * _N, _F)
    idx3 = jnp.stack(
        [pool_idx[:, 0].reshape(2, 128), pool_idx[:, 1].reshape(2, 128)], axis=1
    )
    idx_all = (
        idx3[None] + (jnp.arange(_B, dtype=jnp.int32) * _N)[:, None, None, None]
    ).reshape(_W, 2, 128)
    big = pl.pallas_call(
        _copy_body,
        grid=(_B, 2),
        in_specs=[pl.BlockSpec((1, _N // 2, _F), lambda b, c: (b, c, 0))],
        out_specs=pl.BlockSpec((1, _N // 2, _F), lambda b, c: (b, c, 0)),
        out_shape=jax.ShapeDtypeStruct((_B, _NO, _F), jnp.float32),
    )(inputs)
    add_feat = _pool_call(x_flat, idx_all).reshape(_B, _E, _F)
    return lax.dynamic_update_slice(big, add_feat, (0, _N, 0))


# speedup vs baseline: 1.0030x; 1.0030x over previous
import functools

import jax
import jax.numpy as jnp
from jax import lax
from jax.experimental import pallas as pl
from jax.experimental.pallas import tpu as pltpu
from jax.experimental.pallas import tpu_sc as plsc

_B, _N, _F = 16, 10000, 128
_E = 256
_NO = _N + _E
_HB = _NO // 2
_W = 32
_PW = (_B * _E) // _W


def _pool_body(x_hbm, idx_hbm, out_hbm, bufa, bufb, bufc, idx_v, psem, ssem):
    w = lax.axis_index("s") * 2 + lax.axis_index("c")
    pltpu.sync_copy(idx_hbm.at[w], idx_v)
    pltpu.async_copy(x_hbm.at[idx_v.at[0]], bufa, psem)
    pltpu.async_copy(x_hbm.at[idx_v.at[1]], bufb, psem)
    for _ in range(2):
        pltpu.make_async_copy(x_hbm.at[pl.ds(0, _PW), :], bufa, psem).wait()

    half = _PW // 2

    def _avg_span(lo):
        def _pair_avg(i, carry):
            e = lo + 2 * i
            for r_ in range(2):
                for c_ in range(8):
                    sl = pl.ds(c_ * 16, 16)
                    bufc[e + r_, sl] = 0.5 * (bufa[e + r_, sl] + bufb[e + r_, sl])
            return carry

        lax.fori_loop(0, half // 2, _pair_avg, 0)

    _avg_span(0)
    s0 = pltpu.make_async_copy(
        bufc.at[pl.ds(0, half), :], out_hbm.at[pl.ds(_PW * w, half), :], ssem
    )
    s0.start()
    _avg_span(half)
    s1 = pltpu.make_async_copy(
        bufc.at[pl.ds(half, half), :],
        out_hbm.at[pl.ds(_PW * w + half, half), :],
        ssem,
    )
    s1.start()
    s0.wait()
    s1.wait()


_pool_call = functools.partial(
    pl.kernel,
    out_type=jax.ShapeDtypeStruct((_B * _E, _F), jnp.float32),
    mesh=plsc.VectorSubcoreMesh(core_axis_name="c", subcore_axis_name="s"),
    scratch_types=[
        pltpu.VMEM((_PW, _F), jnp.float32),
        pltpu.VMEM((_PW, _F), jnp.float32),
        pltpu.VMEM((_PW, _F), jnp.float32),
        pltpu.VMEM((2, 128), jnp.int32),
        pltpu.SemaphoreType.DMA,
        pltpu.SemaphoreType.DMA,
    ],
)(_pool_body)


def _copy_body(in_ref, out_ref):
    out_ref[...] = in_ref[...]


def kernel(inputs, pool_idx):
    x_flat = inputs.reshape(_B * _N, _F)
    idx3 = jnp.stack(
        [pool_idx[:, 0].reshape(2, 128), pool_idx[:, 1].reshape(2, 128)], axis=1
    )
    idx_all = (
        idx3[None] + (jnp.arange(_B, dtype=jnp.int32) * _N)[:, None, None, None]
    ).reshape(_W, 2, 128)
    big = pl.pallas_call(
        _copy_body,
        grid=(_B, 2),
        in_specs=[pl.BlockSpec((1, _N // 2, _F), lambda b, c: (b, c, 0))],
        out_specs=pl.BlockSpec((1, _N // 2, _F), lambda b, c: (b, c, 0)),
        out_shape=jax.ShapeDtypeStruct((_B, _NO, _F), jnp.float32),
    )(inputs)
    add_feat = _pool_call(x_flat, idx_all).reshape(_B, _E, _F)
    return lax.dynamic_update_slice(big, add_feat, (0, _N, 0))
